# baseline (speedup 1.0000x reference)
import jax
import jax.numpy as jnp
from jax import lax
from jax.experimental import pallas as pl

BATCH = 4
SEQ = 2048
DIM = 1024
TAB = 512

S_BLK = 1024
B_BLK = 2


def kernel(embeddings, positions, pos_table):
    n_blk = SEQ // S_BLK
    pos3 = positions.reshape(n_blk, 1, S_BLK)

    def body(pos_ref, tab_ref, emb_ref, out_ref):
        pos = pos_ref[0, 0, :]
        onehot = (
            pos[:, None]
            == lax.broadcasted_iota(jnp.int32, (S_BLK, TAB), 1)
        ).astype(jnp.bfloat16)
        tab = tab_ref[...]
        tab_hi = tab.astype(jnp.bfloat16)
        tab_lo = (tab - tab_hi.astype(jnp.float32)).astype(jnp.bfloat16)
        dn = (((1,), (0,)), ((), ()))
        rows = lax.dot_general(
            onehot, tab_hi, dn, preferred_element_type=jnp.float32,
        ) + lax.dot_general(
            onehot, tab_lo, dn, preferred_element_type=jnp.float32,
        )
        out_ref[...] = emb_ref[...] + rows[None, :, :]

    return pl.pallas_call(
        body,
        grid=(n_blk, BATCH // B_BLK),
        in_specs=[
            pl.BlockSpec((1, 1, S_BLK), lambda i, j: (i, 0, 0)),
            pl.BlockSpec((TAB, DIM), lambda i, j: (0, 0)),
            pl.BlockSpec((B_BLK, S_BLK, DIM), lambda i, j: (j, i, 0)),
        ],
        out_specs=pl.BlockSpec((B_BLK, S_BLK, DIM), lambda i, j: (j, i, 0)),
        out_shape=jax.ShapeDtypeStruct((BATCH, SEQ, DIM), jnp.float32),
    )(pos3, pos_table, embeddings)


# speedup vs baseline: 1.0919x; 1.0919x over previous
import jax
import jax.numpy as jnp
from jax import lax
from jax.experimental import pallas as pl

BATCH = 4
SEQ = 2048
DIM = 1024
TAB = 512

S_BLK = 512


def kernel(embeddings, positions, pos_table):
    n_blk = SEQ // S_BLK
    pos3 = positions.reshape(n_blk, 1, S_BLK)

    def body(pos_ref, tab_ref, emb_ref, out_ref):
        pos = pos_ref[0, 0, :]
        onehot = (
            pos[:, None]
            == lax.broadcasted_iota(jnp.int32, (S_BLK, TAB), 1)
        ).astype(jnp.bfloat16)
        tab = tab_ref[...]
        tab_hi = tab.astype(jnp.bfloat16)
        tab_lo = (tab - tab_hi.astype(jnp.float32)).astype(jnp.bfloat16)
        dn = (((1,), (0,)), ((), ()))
        rows = lax.dot_general(
            onehot, tab_hi, dn, preferred_element_type=jnp.float32,
        ) + lax.dot_general(
            onehot, tab_lo, dn, preferred_element_type=jnp.float32,
        )
        out_ref[...] = emb_ref[...] + rows[None, :, :]

    return pl.pallas_call(
        body,
        grid=(n_blk,),
        in_specs=[
            pl.BlockSpec((1, 1, S_BLK), lambda i: (i, 0, 0)),
            pl.BlockSpec((TAB, DIM), lambda i: (0, 0)),
            pl.BlockSpec((BATCH, S_BLK, DIM), lambda i: (0, i, 0)),
        ],
        out_specs=pl.BlockSpec((BATCH, S_BLK, DIM), lambda i: (0, i, 0)),
        out_shape=jax.ShapeDtypeStruct((BATCH, SEQ, DIM), jnp.float32),
    )(pos3, pos_table, embeddings)
